# baseline (speedup 1.0000x reference)
import functools

import jax
import jax.numpy as jnp
from jax import lax
from jax.experimental import pallas as pl
from jax.experimental.pallas import tpu as pltpu
from jax.experimental.pallas import tpu_sc as plsc

_NU = 4000
_NI = 6000
_N = _NU + _NI
_D = 256
_E = 160000
_NW = 32
_K = 64
_PAD = 5 * _K
_FB = 4
_ZR = 32


def _make_layer():
    mesh = plsc.VectorSubcoreMesh(core_axis_name="c", subcore_axis_name="s")

    @functools.partial(
        pl.kernel,
        out_type=jax.ShapeDtypeStruct((_N * _D,), jnp.float32),
        mesh=mesh,
        scratch_types=[
            pltpu.VMEM((16,), jnp.int32),
            pltpu.VMEM((_K,), jnp.int32),
            pltpu.VMEM((_K,), jnp.int32),
            pltpu.VMEM((_K,), jnp.int32),
            pltpu.VMEM((_K,), jnp.int32),
            pltpu.VMEM((_K,), jnp.float32),
            pltpu.VMEM((_K,), jnp.float32),
            pltpu.VMEM((_K, _D), jnp.float32),
            pltpu.VMEM((_K, _D), jnp.float32),
            pltpu.VMEM((_D,), jnp.float32),
            pltpu.VMEM((_FB * _D,), jnp.float32),
            pltpu.VMEM((_ZR * _D,), jnp.float32),
            pltpu.SemaphoreType.DMA,
            pltpu.SemaphoreType.DMA,
            pltpu.SemaphoreType.DMA,
            pltpu.SemaphoreType.DMA,
            pltpu.SemaphoreType.DMA,
        ],
    )
    def layer(bounds_hbm, src_hbm, dst_hbm, w_hbm, emb_hbm, out_hbm,
              bnds, srcA, srcB, dstA, dstB, wA, wB, rowsA, rowsB,
              acc, fbuf, zblk, esemA, esemB, gsemA, gsemB, fsem):
        cid = lax.axis_index("c")
        sid = lax.axis_index("s")
        wid = sid * 2 + cid

        pltpu.sync_copy(bounds_hbm.at[wid], bnds)
        bv = bnds[...]
        e_lo = bv[0]
        e_hi = bv[1]
        n_lo = bv[2]
        n_hi = bv[3]

        zeros16 = jnp.zeros((16,), jnp.float32)
        for j in range(_ZR * _D // 16):
            zblk[pl.ds(16 * j, 16)] = zeros16
        for j in range(_D // 16):
            acc[pl.ds(16 * j, 16)] = zeros16

        def zfill(s, c):
            nf = c // _ZR

            def zf(k, carry):
                pltpu.sync_copy(zblk,
                                out_hbm.at[pl.ds((s + _ZR * k) * _D, _ZR * _D)])
                return carry

            lax.fori_loop(0, nf, zf, 0)

            def zt(r, carry):
                pltpu.sync_copy(
                    zblk.at[pl.ds(0, _D)],
                    out_hbm.at[pl.ds((s + _ZR * nf + r) * _D, _D)])
                return carry

            lax.fori_loop(0, c - _ZR * nf, zt, 0)

        def do_flush(cd, lw, f):
            zfill(lw + 1, cd - lw - 1)

            @pl.when(f >= _FB)
            def _():
                pltpu.make_async_copy(fbuf.at[pl.ds(0, _D)],
                                      out_hbm.at[pl.ds(0, _D)], fsem).wait()

            ab = (f % _FB) * _D
            for j in range(_D // 16):
                fbuf[pl.ds(ab + 16 * j, 16)] = acc[pl.ds(16 * j, 16)]
                acc[pl.ds(16 * j, 16)] = zeros16
            pltpu.async_copy(fbuf.at[pl.ds(ab, _D)],
                             out_hbm.at[pl.ds(cd * _D, _D)], fsem)
            return cd, f + 1

        def edata_start(eb, sb, db, wb2, sem):
            pltpu.async_copy(src_hbm.at[pl.ds(eb, _K)], sb, sem)
            pltpu.async_copy(dst_hbm.at[pl.ds(eb, _K)], db, sem)
            pltpu.async_copy(w_hbm.at[pl.ds(eb, _K)], wb2, sem)

        def edata_wait(eb, sb, db, wb2, sem):
            pltpu.make_async_copy(src_hbm.at[pl.ds(eb, _K)], sb, sem).wait()
            pltpu.make_async_copy(dst_hbm.at[pl.ds(eb, _K)], db, sem).wait()
            pltpu.make_async_copy(w_hbm.at[pl.ds(eb, _K)], wb2, sem).wait()

        def gather_start(eb, sb, rows, sem):
            @pl.when(eb < e_hi)
            def _():
                pltpu.async_copy(emb_hbm.at[sb], rows, sem)

        def gather_wait(eb, sb, rows, sem):
            @pl.when(eb < e_hi)
            def _():
                pltpu.make_async_copy(emb_hbm.at[sb], rows, sem).wait()

        def process_block(eb, rows, db, wb2, carry):
            def grp(q, carry):
                i0 = 16 * q
                dvec = db[pl.ds(i0, 16)]
                wvec = wb2[pl.ds(i0, 16)]
                gv = (eb + i0) + lax.iota(jnp.int32, 16)
                inr = jnp.logical_and(gv >= e_lo, gv < e_hi)
                wvec = jnp.where(inr, wvec, 0.0)
                for l in range(16):
                    i = i0 + l
                    d = dvec[l]
                    w = wvec[l]
                    cd, lw, f = carry
                    guard = jnp.logical_and(d != cd,
                                            jnp.logical_and(cd >= n_lo,
                                                            cd < n_hi))
                    lw, f = lax.cond(guard, do_flush,
                                     lambda cd, lw, f: (lw, f), cd, lw, f)
                    for j in range(_D // 16):
                        v = rows[i, pl.ds(16 * j, 16)] * w
                        plsc.addupdate(acc.at[pl.ds(16 * j, 16)], v)
                    carry = (d, lw, f)
                return carry

            return lax.fori_loop(0, _K // 16, grp, carry)

        base0 = (e_lo // _K) * _K
        nblk = jnp.where(e_lo < e_hi, (e_hi - base0 + _K - 1) // _K, 0)
        nsuper = (nblk + 1) // 2

        def blk_at(b):
            return base0 + b * _K

        def blk_body(b, carry):
            eb = blk_at(b)
            edata_start(eb, srcA, dstA, wA, esemA)
            edata_wait(eb, srcA, dstA, wA, esemA)
            gather_start(eb, srcA, rowsA, gsemA)
            gather_wait(eb, srcA, rowsA, gsemA)
            return process_block(eb, rowsA, dstA, wA, carry)

        cd, lw, f = lax.fori_loop(0, nblk, blk_body,
                                  (jnp.int32(-1), n_lo - 1, jnp.int32(0)))

        fguard = jnp.logical_and(cd >= n_lo, cd < n_hi)
        lw, f = lax.cond(fguard, do_flush, lambda cd, lw, f: (lw, f),
                         cd, lw, f)
        zfill(lw + 1, n_hi - 1 - lw)

        def drain(k, carry):
            pltpu.make_async_copy(fbuf.at[pl.ds(0, _D)],
                                  out_hbm.at[pl.ds(0, _D)], fsem).wait()
            return carry

        lax.fori_loop(0, jnp.minimum(f, _FB), drain, 0)

    return layer


_layer = _make_layer()


def _norm_body(x_ref, o_ref):
    x = x_ref[...]
    s = jnp.sum(x * x, axis=1, keepdims=True)
    o_ref[...] = x / jnp.maximum(jnp.sqrt(s), 1e-12)


def _l2n(x, blk):
    m = x.shape[0]
    return pl.pallas_call(
        _norm_body,
        out_shape=jax.ShapeDtypeStruct(x.shape, x.dtype),
        grid=(m // blk,),
        in_specs=[pl.BlockSpec((blk, _D), lambda i: (i, 0))],
        out_specs=pl.BlockSpec((blk, _D), lambda i: (i, 0)),
    )(x)


def _final_body(a_ref, b_ref, c_ref, d_ref, o_ref):
    x = (a_ref[...] + b_ref[...] + c_ref[...] + d_ref[...]) * 0.25
    s = jnp.sum(x * x, axis=1, keepdims=True)
    o_ref[...] = x / jnp.maximum(jnp.sqrt(s), 1e-12)


def _finalize(a, b, c, d, blk=2000):
    spec = pl.BlockSpec((blk, _D), lambda i: (i, 0))
    return pl.pallas_call(
        _final_body,
        out_shape=jax.ShapeDtypeStruct((_N, _D), jnp.float32),
        grid=(_N // blk,),
        in_specs=[spec, spec, spec, spec],
        out_specs=spec,
    )(a, b, c, d)


def kernel(edge_index, edge_weight, user_emb_w, item_emb_w):
    src = edge_index[0].astype(jnp.int32)
    dst = edge_index[1].astype(jnp.int32)

    order = jnp.argsort(dst)
    srcs = src[order]
    dsts = dst[order]
    ws = edge_weight[order]
    srcp = jnp.concatenate([srcs, jnp.zeros((_PAD,), jnp.int32)])
    dstp = jnp.concatenate([dsts, jnp.full((_PAD,), _N, jnp.int32)])
    wp = jnp.concatenate([ws, jnp.zeros((_PAD,), jnp.float32)])

    starts = jnp.arange(_NW, dtype=jnp.int32) * (_E // _NW)
    nlo = jnp.where(jnp.arange(_NW) == 0, 0, dsts[starts]).astype(jnp.int32)
    nhi = jnp.concatenate([nlo[1:], jnp.array([_N], jnp.int32)])
    elo = jnp.searchsorted(dsts, nlo, side="left").astype(jnp.int32)
    ehi = jnp.concatenate([elo[1:], jnp.array([_E], jnp.int32)])
    zeros = jnp.zeros((_NW,), jnp.int32)
    bounds = jnp.stack([elo, ehi, nlo, nhi] + [zeros] * 12, axis=1)

    emb0 = jnp.concatenate([_l2n(user_emb_w, 2000), _l2n(item_emb_w, 2000)],
                           axis=0)
    embs = [emb0]
    e = emb0
    for _ in range(3):
        e = _layer(bounds, srcp, dstp, wp, e).reshape(_N, _D)
        embs.append(e)
    final = _finalize(*embs)
    return final[:_NU], final[_NU:]


# speedup vs baseline: 1.2240x; 1.2240x over previous
import functools

import jax
import jax.numpy as jnp
from jax import lax
from jax.experimental import pallas as pl
from jax.experimental.pallas import tpu as pltpu
from jax.experimental.pallas import tpu_sc as plsc

_NU = 4000
_NI = 6000
_N = _NU + _NI
_D = 256
_E = 160000
_NW = 32
_K = 64
_PAD = 5 * _K
_FB = 4
_ZR = 32


def _make_layer():
    mesh = plsc.VectorSubcoreMesh(core_axis_name="c", subcore_axis_name="s")

    @functools.partial(
        pl.kernel,
        out_type=jax.ShapeDtypeStruct((_N * _D,), jnp.float32),
        mesh=mesh,
        scratch_types=[
            pltpu.VMEM((16,), jnp.int32),
            pltpu.VMEM((_K,), jnp.int32),
            pltpu.VMEM((_K,), jnp.int32),
            pltpu.VMEM((_K,), jnp.int32),
            pltpu.VMEM((_K,), jnp.int32),
            pltpu.VMEM((_K,), jnp.float32),
            pltpu.VMEM((_K,), jnp.float32),
            pltpu.VMEM((_K, _D), jnp.float32),
            pltpu.VMEM((_K, _D), jnp.float32),
            pltpu.VMEM((_D,), jnp.float32),
            pltpu.VMEM((_FB * _D,), jnp.float32),
            pltpu.VMEM((_ZR * _D,), jnp.float32),
            pltpu.SemaphoreType.DMA,
            pltpu.SemaphoreType.DMA,
            pltpu.SemaphoreType.DMA,
            pltpu.SemaphoreType.DMA,
            pltpu.SemaphoreType.DMA,
        ],
    )
    def layer(bounds_hbm, src_hbm, dst_hbm, w_hbm, emb_hbm, out_hbm,
              bnds, srcA, srcB, dstA, dstB, wA, wB, rowsA, rowsB,
              acc, fbuf, zblk, esemA, esemB, gsemA, gsemB, fsem):
        cid = lax.axis_index("c")
        sid = lax.axis_index("s")
        wid = sid * 2 + cid

        pltpu.sync_copy(bounds_hbm.at[wid], bnds)
        bv = bnds[...]
        e_lo = bv[0]
        e_hi = bv[1]
        n_lo = bv[2]
        n_hi = bv[3]

        zeros16 = jnp.zeros((16,), jnp.float32)
        for j in range(_ZR * _D // 16):
            zblk[pl.ds(16 * j, 16)] = zeros16
        for j in range(_D // 16):
            acc[pl.ds(16 * j, 16)] = zeros16

        def zfill(s, c):
            nf = c // _ZR

            def zf(k, carry):
                pltpu.sync_copy(zblk,
                                out_hbm.at[pl.ds((s + _ZR * k) * _D, _ZR * _D)])
                return carry

            lax.fori_loop(0, nf, zf, 0)

            def zt(r, carry):
                pltpu.sync_copy(
                    zblk.at[pl.ds(0, _D)],
                    out_hbm.at[pl.ds((s + _ZR * nf + r) * _D, _D)])
                return carry

            lax.fori_loop(0, c - _ZR * nf, zt, 0)

        def do_flush(cd, lw, f):
            zfill(lw + 1, cd - lw - 1)
            pltpu.sync_copy(acc, out_hbm.at[pl.ds(cd * _D, _D)])
            for j in range(_D // 16):
                acc[pl.ds(16 * j, 16)] = zeros16
            return cd, f

        def edata_start(eb, sb, db, wb2, sem):
            pltpu.async_copy(src_hbm.at[pl.ds(eb, _K)], sb, sem)
            pltpu.async_copy(dst_hbm.at[pl.ds(eb, _K)], db, sem)
            pltpu.async_copy(w_hbm.at[pl.ds(eb, _K)], wb2, sem)

        def edata_wait(eb, sb, db, wb2, sem):
            pltpu.make_async_copy(src_hbm.at[pl.ds(eb, _K)], sb, sem).wait()
            pltpu.make_async_copy(dst_hbm.at[pl.ds(eb, _K)], db, sem).wait()
            pltpu.make_async_copy(w_hbm.at[pl.ds(eb, _K)], wb2, sem).wait()

        def gather_start(eb, sb, rows, sem):
            @pl.when(eb < e_hi)
            def _():
                pltpu.async_copy(emb_hbm.at[sb], rows, sem)

        def gather_wait(eb, sb, rows, sem):
            @pl.when(eb < e_hi)
            def _():
                pltpu.make_async_copy(emb_hbm.at[sb], rows, sem).wait()

        def process_block(eb, rows, db, wb2, carry):
            def grp(q, carry):
                i0 = 16 * q
                dvec = db[pl.ds(i0, 16)]
                wvec = wb2[pl.ds(i0, 16)]
                gv = (eb + i0) + lax.iota(jnp.int32, 16)
                inr = jnp.logical_and(gv >= e_lo, gv < e_hi)
                wvec = jnp.where(inr, wvec, 0.0)
                for l in range(16):
                    i = i0 + l
                    d = dvec[l]
                    w = wvec[l]
                    cd, lw, f = carry
                    guard = jnp.logical_and(d != cd,
                                            jnp.logical_and(cd >= n_lo,
                                                            cd < n_hi))
                    lw, f = lax.cond(guard, do_flush,
                                     lambda cd, lw, f: (lw, f), cd, lw, f)
                    for j in range(_D // 16):
                        v = rows[i, pl.ds(16 * j, 16)] * w
                        plsc.addupdate(acc.at[pl.ds(16 * j, 16)], v)
                    carry = (d, lw, f)
                return carry

            return lax.fori_loop(0, _K // 16, grp, carry)

        base0 = (e_lo // _K) * _K
        nblk = jnp.where(e_lo < e_hi, (e_hi - base0 + _K - 1) // _K, 0)
        nsuper = (nblk + 1) // 2

        def blk_at(b):
            return base0 + b * _K

        def blk_body(b, carry):
            eb = blk_at(b)
            edata_start(eb, srcA, dstA, wA, esemA)
            edata_wait(eb, srcA, dstA, wA, esemA)
            gather_start(eb, srcA, rowsA, gsemA)
            gather_wait(eb, srcA, rowsA, gsemA)
            return process_block(eb, rowsA, dstA, wA, carry)

        cd, lw, f = lax.fori_loop(0, nblk, blk_body,
                                  (jnp.int32(-1), n_lo - 1, jnp.int32(0)))

        fguard = jnp.logical_and(cd >= n_lo, cd < n_hi)
        lw, f = lax.cond(fguard, do_flush, lambda cd, lw, f: (lw, f),
                         cd, lw, f)
        zfill(lw + 1, n_hi - 1 - lw)

        def drain(k, carry):
            pltpu.make_async_copy(fbuf.at[pl.ds(0, _D)],
                                  out_hbm.at[pl.ds(0, _D)], fsem).wait()
            return carry

        lax.fori_loop(0, jnp.minimum(f, _FB), drain, 0)

    return layer


_layer = _make_layer()


def _norm_body(x_ref, o_ref):
    x = x_ref[...]
    s = jnp.sum(x * x, axis=1, keepdims=True)
    o_ref[...] = x / jnp.maximum(jnp.sqrt(s), 1e-12)


def _l2n(x, blk):
    m = x.shape[0]
    return pl.pallas_call(
        _norm_body,
        out_shape=jax.ShapeDtypeStruct(x.shape, x.dtype),
        grid=(m // blk,),
        in_specs=[pl.BlockSpec((blk, _D), lambda i: (i, 0))],
        out_specs=pl.BlockSpec((blk, _D), lambda i: (i, 0)),
    )(x)


def _final_body(a_ref, b_ref, c_ref, d_ref, o_ref):
    x = (a_ref[...] + b_ref[...] + c_ref[...] + d_ref[...]) * 0.25
    s = jnp.sum(x * x, axis=1, keepdims=True)
    o_ref[...] = x / jnp.maximum(jnp.sqrt(s), 1e-12)


def _finalize(a, b, c, d, blk=2000):
    spec = pl.BlockSpec((blk, _D), lambda i: (i, 0))
    return pl.pallas_call(
        _final_body,
        out_shape=jax.ShapeDtypeStruct((_N, _D), jnp.float32),
        grid=(_N // blk,),
        in_specs=[spec, spec, spec, spec],
        out_specs=spec,
    )(a, b, c, d)


def kernel(edge_index, edge_weight, user_emb_w, item_emb_w):
    src = edge_index[0].astype(jnp.int32)
    dst = edge_index[1].astype(jnp.int32)

    order = jnp.argsort(dst)
    srcs = src[order]
    dsts = dst[order]
    ws = edge_weight[order]
    srcp = jnp.concatenate([srcs, jnp.zeros((_PAD,), jnp.int32)])
    dstp = jnp.concatenate([dsts, jnp.full((_PAD,), _N, jnp.int32)])
    wp = jnp.concatenate([ws, jnp.zeros((_PAD,), jnp.float32)])

    starts = jnp.arange(_NW, dtype=jnp.int32) * (_E // _NW)
    nlo = jnp.where(jnp.arange(_NW) == 0, 0, dsts[starts]).astype(jnp.int32)
    nhi = jnp.concatenate([nlo[1:], jnp.array([_N], jnp.int32)])
    elo = jnp.searchsorted(dsts, nlo, side="left").astype(jnp.int32)
    ehi = jnp.concatenate([elo[1:], jnp.array([_E], jnp.int32)])
    zeros = jnp.zeros((_NW,), jnp.int32)
    bounds = jnp.stack([elo, ehi, nlo, nhi] + [zeros] * 12, axis=1)

    emb0 = jnp.concatenate([_l2n(user_emb_w, 2000), _l2n(item_emb_w, 2000)],
                           axis=0)
    embs = [emb0]
    e = emb0
    for _ in range(3):
        e = _layer(bounds, srcp, dstp, wp, e).reshape(_N, _D)
        embs.append(e)
    final = _finalize(*embs)
    return final[:_NU], final[_NU:]


# speedup vs baseline: 2.0189x; 1.6494x over previous
import functools

import jax
import jax.numpy as jnp
from jax import lax
from jax.experimental import pallas as pl
from jax.experimental.pallas import tpu as pltpu
from jax.experimental.pallas import tpu_sc as plsc

_NU = 4000
_NI = 6000
_N = _NU + _NI
_D = 256
_E = 160000
_NW = 32
_K = 64
_PAD = 5 * _K
_FB = 4
_ZR = 32


def _make_layer():
    mesh = plsc.VectorSubcoreMesh(core_axis_name="c", subcore_axis_name="s")

    @functools.partial(
        pl.kernel,
        out_type=jax.ShapeDtypeStruct((_N * _D,), jnp.float32),
        mesh=mesh,
        scratch_types=[
            pltpu.VMEM((16,), jnp.int32),
            pltpu.VMEM((_K,), jnp.int32),
            pltpu.VMEM((_K,), jnp.int32),
            pltpu.VMEM((_K,), jnp.int32),
            pltpu.VMEM((_K,), jnp.int32),
            pltpu.VMEM((_K,), jnp.float32),
            pltpu.VMEM((_K,), jnp.float32),
            pltpu.VMEM((_K, _D), jnp.float32),
            pltpu.VMEM((_K, _D), jnp.float32),
            pltpu.VMEM((_D,), jnp.float32),
            pltpu.VMEM((_FB * _D,), jnp.float32),
            pltpu.VMEM((_ZR * _D,), jnp.float32),
            pltpu.SemaphoreType.DMA,
            pltpu.SemaphoreType.DMA,
            pltpu.SemaphoreType.DMA,
            pltpu.SemaphoreType.DMA,
            pltpu.SemaphoreType.DMA,
        ],
    )
    def layer(bounds_hbm, src_hbm, dst_hbm, w_hbm, emb_hbm, out_hbm,
              bnds, srcA, srcB, dstA, dstB, wA, wB, rowsA, rowsB,
              acc, fbuf, zblk, esemA, esemB, gsemA, gsemB, fsem):
        cid = lax.axis_index("c")
        sid = lax.axis_index("s")
        wid = sid * 2 + cid

        pltpu.sync_copy(bounds_hbm.at[wid], bnds)
        bv = bnds[...]
        e_lo = bv[0]
        e_hi = bv[1]
        n_lo = bv[2]
        n_hi = bv[3]

        zeros16 = jnp.zeros((16,), jnp.float32)
        for j in range(_ZR * _D // 16):
            zblk[pl.ds(16 * j, 16)] = zeros16
        for j in range(_D // 16):
            acc[pl.ds(16 * j, 16)] = zeros16

        cnt = n_hi - n_lo
        nzf = cnt // _ZR

        def zf(k, carry):
            pltpu.sync_copy(zblk,
                            out_hbm.at[pl.ds((n_lo + _ZR * k) * _D, _ZR * _D)])
            return carry

        lax.fori_loop(0, nzf, zf, 0)

        def zt(r, carry):
            pltpu.sync_copy(
                zblk.at[pl.ds(0, _D)],
                out_hbm.at[pl.ds((n_lo + _ZR * nzf + r) * _D, _D)])
            return carry

        lax.fori_loop(0, cnt - _ZR * nzf, zt, 0)

        def flush_row(cd):
            pltpu.sync_copy(acc, out_hbm.at[pl.ds(cd * _D, _D)])
            for j in range(_D // 16):
                acc[pl.ds(16 * j, 16)] = zeros16

        def edata_start(eb, sb, db, wb2, sem):
            pltpu.async_copy(src_hbm.at[pl.ds(eb, _K)], sb, sem)
            pltpu.async_copy(dst_hbm.at[pl.ds(eb, _K)], db, sem)
            pltpu.async_copy(w_hbm.at[pl.ds(eb, _K)], wb2, sem)

        def edata_wait(eb, sb, db, wb2, sem):
            pltpu.make_async_copy(src_hbm.at[pl.ds(eb, _K)], sb, sem).wait()
            pltpu.make_async_copy(dst_hbm.at[pl.ds(eb, _K)], db, sem).wait()
            pltpu.make_async_copy(w_hbm.at[pl.ds(eb, _K)], wb2, sem).wait()

        def gather_start(eb, sb, rows, sem):
            @pl.when(eb < e_hi)
            def _():
                pltpu.async_copy(emb_hbm.at[sb], rows, sem)

        def gather_wait(eb, sb, rows, sem):
            @pl.when(eb < e_hi)
            def _():
                pltpu.make_async_copy(emb_hbm.at[sb], rows, sem).wait()

        def process_block(eb, rows, db, wb2, carry):
            def grp(q, carry):
                i0 = 16 * q
                dvec = db[pl.ds(i0, 16)]
                wvec = wb2[pl.ds(i0, 16)]
                gv = (eb + i0) + lax.iota(jnp.int32, 16)
                inr = jnp.logical_and(gv >= e_lo, gv < e_hi)
                wvec = jnp.where(inr, wvec, 0.0)
                for l in range(16):
                    i = i0 + l
                    d = dvec[l]
                    w = wvec[l]
                    cd = carry
                    guard = jnp.logical_and(d != cd,
                                            jnp.logical_and(cd >= n_lo,
                                                            cd < n_hi))

                    @pl.when(guard)
                    def _(cd=cd):
                        flush_row(cd)

                    for j in range(_D // 16):
                        v = rows[i, pl.ds(16 * j, 16)] * w
                        plsc.addupdate(acc.at[pl.ds(16 * j, 16)], v)
                    carry = d
                return carry

            return lax.fori_loop(0, _K // 16, grp, carry)

        base0 = (e_lo // _K) * _K
        nblk = jnp.where(e_lo < e_hi, (e_hi - base0 + _K - 1) // _K, 0)
        nsuper = (nblk + 1) // 2

        def blk_at(b):
            return base0 + b * _K

        def blk_body(b, carry):
            eb = blk_at(b)
            edata_start(eb, srcA, dstA, wA, esemA)
            edata_wait(eb, srcA, dstA, wA, esemA)
            gather_start(eb, srcA, rowsA, gsemA)
            gather_wait(eb, srcA, rowsA, gsemA)
            return process_block(eb, rowsA, dstA, wA, carry)

        cd = lax.fori_loop(0, nblk, blk_body, jnp.int32(-1))

        fguard = jnp.logical_and(cd >= n_lo, cd < n_hi)

        @pl.when(fguard)
        def _():
            flush_row(cd)

    return layer


_layer = _make_layer()


def _norm_body(x_ref, o_ref):
    x = x_ref[...]
    s = jnp.sum(x * x, axis=1, keepdims=True)
    o_ref[...] = x / jnp.maximum(jnp.sqrt(s), 1e-12)


def _l2n(x, blk):
    m = x.shape[0]
    return pl.pallas_call(
        _norm_body,
        out_shape=jax.ShapeDtypeStruct(x.shape, x.dtype),
        grid=(m // blk,),
        in_specs=[pl.BlockSpec((blk, _D), lambda i: (i, 0))],
        out_specs=pl.BlockSpec((blk, _D), lambda i: (i, 0)),
    )(x)


def _final_body(a_ref, b_ref, c_ref, d_ref, o_ref):
    x = (a_ref[...] + b_ref[...] + c_ref[...] + d_ref[...]) * 0.25
    s = jnp.sum(x * x, axis=1, keepdims=True)
    o_ref[...] = x / jnp.maximum(jnp.sqrt(s), 1e-12)


def _finalize(a, b, c, d, blk=2000):
    spec = pl.BlockSpec((blk, _D), lambda i: (i, 0))
    return pl.pallas_call(
        _final_body,
        out_shape=jax.ShapeDtypeStruct((_N, _D), jnp.float32),
        grid=(_N // blk,),
        in_specs=[spec, spec, spec, spec],
        out_specs=spec,
    )(a, b, c, d)


def kernel(edge_index, edge_weight, user_emb_w, item_emb_w):
    src = edge_index[0].astype(jnp.int32)
    dst = edge_index[1].astype(jnp.int32)

    order = jnp.argsort(dst)
    srcs = src[order]
    dsts = dst[order]
    ws = edge_weight[order]
    srcp = jnp.concatenate([srcs, jnp.zeros((_PAD,), jnp.int32)])
    dstp = jnp.concatenate([dsts, jnp.full((_PAD,), _N, jnp.int32)])
    wp = jnp.concatenate([ws, jnp.zeros((_PAD,), jnp.float32)])

    starts = jnp.arange(_NW, dtype=jnp.int32) * (_E // _NW)
    nlo = jnp.where(jnp.arange(_NW) == 0, 0, dsts[starts]).astype(jnp.int32)
    nhi = jnp.concatenate([nlo[1:], jnp.array([_N], jnp.int32)])
    elo = jnp.searchsorted(dsts, nlo, side="left").astype(jnp.int32)
    ehi = jnp.concatenate([elo[1:], jnp.array([_E], jnp.int32)])
    zeros = jnp.zeros((_NW,), jnp.int32)
    bounds = jnp.stack([elo, ehi, nlo, nhi] + [zeros] * 12, axis=1)

    emb0 = jnp.concatenate([_l2n(user_emb_w, 2000), _l2n(item_emb_w, 2000)],
                           axis=0)
    embs = [emb0]
    e = emb0
    for _ in range(3):
        e = _layer(bounds, srcp, dstp, wp, e).reshape(_N, _D)
        embs.append(e)
    final = _finalize(*embs)
    return final[:_NU], final[_NU:]
